# baseline (speedup 1.0000x reference)
import functools

import jax
import jax.numpy as jnp
from jax import lax
from jax.experimental import pallas as pl
from jax.experimental.pallas import tpu as pltpu
from jax.experimental.pallas import tpu_sc as plsc

V_SIZE = 1000000
EMB_SIZE = 64
BATCH = 16384

NC = 2
NS = 16
LANES = 16
NW = NC * NS
B_PER_W = BATCH // NW
CHUNK = 128
N_CHUNKS = B_PER_W // CHUNK
ROW = 2 * EMB_SIZE


def _sc_partials(focus, context, emb2):
  mesh = plsc.VectorSubcoreMesh(core_axis_name="c", subcore_axis_name="s")

  @functools.partial(
      pl.kernel,
      out_type=jax.ShapeDtypeStruct((NW, LANES), jnp.float32),
      mesh=mesh,
      scratch_types=[
          pltpu.VMEM((B_PER_W,), jnp.int32),
          pltpu.VMEM((B_PER_W,), jnp.int32),
          pltpu.VMEM((B_PER_W,), jnp.int32),
          pltpu.VMEM((B_PER_W,), jnp.int32),
          pltpu.VMEM((2, CHUNK, ROW), jnp.float32),
          pltpu.VMEM((2, CHUNK, ROW), jnp.float32),
          pltpu.VMEM((LANES,), jnp.float32),
          pltpu.SemaphoreType.DMA,
          pltpu.SemaphoreType.DMA,
      ],
  )
  def body(focus_hbm, ctx_hbm, emb_hbm, out_hbm,
           row_f, row_c, off_f, off_c, bf, bc, partial_v, sem0, sem1):
    wid = lax.axis_index("s") * NC + lax.axis_index("c")
    base = wid * B_PER_W

    pltpu.sync_copy(focus_hbm.at[pl.ds(base, B_PER_W)], row_f)
    pltpu.sync_copy(ctx_hbm.at[pl.ds(base, B_PER_W)], row_c)

    def split_body(g, _):
      sl = pl.ds(g * LANES, LANES)
      for idx_ref, off_ref in ((row_f, off_f), (row_c, off_c)):
        raw = idx_ref[sl]
        off_ref[sl] = (raw & 1) << 6
        idx_ref[sl] = raw >> 1
      return 0

    lax.fori_loop(0, B_PER_W // LANES, split_body, 0)

    n_sub = EMB_SIZE // LANES
    sems = (sem0, sem1)

    def start_chunk(c):
      slot = c & 1
      sl = pl.ds(c * CHUNK, CHUNK)
      return (
          pltpu.async_copy(emb_hbm.at[row_f.at[sl]], bf.at[slot], sems[slot]),
          pltpu.async_copy(emb_hbm.at[row_c.at[sl]], bc.at[slot], sems[slot]),
      )

    def make_group_body(slot, coff):
      def group_body(g, accs):
        base_i = g * LANES
        ofv = off_f[pl.ds(coff + base_i, LANES)]
        ocv = off_c[pl.ds(coff + base_i, LANES)]
        accs = list(accs)
        for l in range(LANES):
          i = base_i + l
          of = ofv[l]
          oc = ocv[l]
          for j in range(n_sub):
            accs[j] = accs[j] + (bf[slot, i, pl.ds(of + j * LANES, LANES)]
                                 * bc[slot, i, pl.ds(oc + j * LANES, LANES)])
        return tuple(accs)
      return group_body

    zero = jnp.zeros((LANES,), jnp.float32)
    accs = (zero,) * n_sub
    pending = start_chunk(0)
    for c in range(N_CHUNKS):
      for cp in pending:
        cp.wait()
      if c + 1 < N_CHUNKS:
        pending = start_chunk(c + 1)
      accs = lax.fori_loop(0, CHUNK // LANES,
                           make_group_body(c & 1, c * CHUNK), accs)
    total = accs[0]
    for j in range(1, n_sub):
      total = total + accs[j]
    partial_v[...] = total
    pltpu.sync_copy(partial_v, out_hbm.at[wid])

  return body(focus, context, emb2)


def _finalize(partials):
  def tc_body(p_ref, o_ref):
    s = jnp.sum(p_ref[...])
    ls = jnp.minimum(s, 0.0) - jnp.log(1.0 + jnp.exp(-jnp.abs(s)))
    o_ref[...] = jnp.reshape(ls, (1, 1))

  return pl.pallas_call(
      tc_body,
      out_shape=jax.ShapeDtypeStruct((1, 1), jnp.float32),
  )(partials)


@jax.jit
def kernel(focus, context, emb):
  emb2 = jnp.reshape(emb, (V_SIZE // 2, ROW))
  partials = _sc_partials(focus, context, emb2)
  return _finalize(partials)


# speedup vs baseline: 1.7131x; 1.7131x over previous
import functools

import jax
import jax.numpy as jnp
from jax import lax
from jax.experimental import pallas as pl
from jax.experimental.pallas import tpu as pltpu
from jax.experimental.pallas import tpu_sc as plsc

V_SIZE = 1000000
EMB_SIZE = 64
BATCH = 16384

NC = 2
NS = 16
LANES = 16
NW = NC * NS
B_PER_W = BATCH // NW
CHUNK = 16
N_CHUNKS = B_PER_W // CHUNK


def _sc_partials(focus, context, emb):
  mesh = plsc.VectorSubcoreMesh(core_axis_name="c", subcore_axis_name="s")

  @functools.partial(
      pl.kernel,
      out_type=jax.ShapeDtypeStruct((NW, LANES), jnp.float32),
      mesh=mesh,
      scratch_types=[
          pltpu.VMEM((B_PER_W,), jnp.int32),
          pltpu.VMEM((B_PER_W,), jnp.int32),
          pltpu.VMEM((2, CHUNK, EMB_SIZE), jnp.float32),
          pltpu.VMEM((2, CHUNK, EMB_SIZE), jnp.float32),
          pltpu.VMEM((LANES,), jnp.float32),
          pltpu.SemaphoreType.DMA,
          pltpu.SemaphoreType.DMA,
      ],
  )
  def body(focus_hbm, ctx_hbm, emb_hbm, out_hbm,
           idx_f, idx_c, bf, bc, partial_v, sem0, sem1):
    wid = lax.axis_index("s") * NC + lax.axis_index("c")
    base = wid * B_PER_W

    pltpu.sync_copy(focus_hbm.at[pl.ds(base, B_PER_W)], idx_f)
    pltpu.sync_copy(ctx_hbm.at[pl.ds(base, B_PER_W)], idx_c)

    n_sub = EMB_SIZE // LANES
    sems = (sem0, sem1)

    def start_chunk(c, slot):
      rv_f = idx_f[pl.ds(c * CHUNK, CHUNK)]
      rv_c = idx_c[pl.ds(c * CHUNK, CHUNK)]
      for l in range(CHUNK):
        pltpu.async_copy(emb_hbm.at[rv_f[l]], bf.at[slot, l], sems[slot])
        pltpu.async_copy(emb_hbm.at[rv_c[l]], bc.at[slot, l], sems[slot])

    def wait_chunk(slot):
      dummy = emb_hbm.at[0]
      for l in range(CHUNK):
        pltpu.make_async_copy(dummy, bf.at[slot, l], sems[slot]).wait()
        pltpu.make_async_copy(dummy, bc.at[slot, l], sems[slot]).wait()

    def compute_chunk(slot, accs):
      accs = list(accs)
      for l in range(CHUNK):
        for j in range(n_sub):
          accs[j] = accs[j] + (bf[slot, l, pl.ds(j * LANES, LANES)]
                               * bc[slot, l, pl.ds(j * LANES, LANES)])
      return tuple(accs)

    start_chunk(0, 0)
    start_chunk(1, 1)

    def pair_body(g, accs):
      c0 = g * 2
      wait_chunk(0)
      @pl.when(c0 + 2 < N_CHUNKS)
      def _():
        start_chunk(c0 + 2, 0)
      accs = compute_chunk(0, accs)
      wait_chunk(1)
      @pl.when(c0 + 3 < N_CHUNKS)
      def _():
        start_chunk(c0 + 3, 1)
      accs = compute_chunk(1, accs)
      return accs

    zero = jnp.zeros((LANES,), jnp.float32)
    accs = lax.fori_loop(0, N_CHUNKS // 2, pair_body, (zero,) * n_sub)

    total = accs[0]
    for j in range(1, n_sub):
      total = total + accs[j]
    partial_v[...] = total
    pltpu.sync_copy(partial_v, out_hbm.at[wid])

  return body(focus, context, emb)


def _finalize(partials):
  def tc_body(p_ref, o_ref):
    s = jnp.sum(p_ref[...])
    ls = jnp.minimum(s, 0.0) - jnp.log(1.0 + jnp.exp(-jnp.abs(s)))
    o_ref[...] = jnp.reshape(ls, (1, 1))

  return pl.pallas_call(
      tc_body,
      out_shape=jax.ShapeDtypeStruct((1, 1), jnp.float32),
  )(partials)


@jax.jit
def kernel(focus, context, emb):
  partials = _sc_partials(focus, context, emb)
  return _finalize(partials)
